# baseline (speedup 1.0000x reference)
import functools

import jax
import jax.numpy as jnp
from jax import lax
from jax.experimental import pallas as pl
from jax.experimental.pallas import tpu as pltpu
from jax.experimental.pallas import tpu_sc as plsc

_NS_DT = 0.1
_NS_STEP = 40
_K = 16
_L = 16
_NC = 2
_NSUB = 16


def _graph_sc(xs, ys, xb, yb, data, labels, tstep):
    nb, tw, nx = data.shape
    nw = _NC * _NSUB
    qpw = nx // nw
    ngroups = nx // _L
    mesh = plsc.VectorSubcoreMesh(core_axis_name="c", subcore_axis_name="s")

    @functools.partial(
        pl.kernel,
        out_type=[
            jax.ShapeDtypeStruct((2 * nb * nx * _K,), jnp.int32),
            jax.ShapeDtypeStruct((nb * nx * tw,), jnp.float32),
            jax.ShapeDtypeStruct((nb * nx * tw,), jnp.float32),
            jax.ShapeDtypeStruct((nb * nx * 3,), jnp.float32),
            jax.ShapeDtypeStruct((nb * nx,), jnp.int32),
        ],
        mesh=mesh,
        compiler_params=pltpu.CompilerParams(needs_layout_passes=False),
        scratch_types=[
            pltpu.VMEM((nx + _L,), jnp.float32),
            pltpu.VMEM((nx + _L,), jnp.float32),
            pltpu.VMEM((nx + _L,), jnp.float32),
            pltpu.VMEM((nx + _L,), jnp.float32),
            pltpu.VMEM((nx + _L,), jnp.float32),
            pltpu.VMEM((nx + _L,), jnp.int32),
            pltpu.VMEM((qpw * _K,), jnp.int32),
            pltpu.VMEM((nb * qpw * _K,), jnp.int32),
            pltpu.VMEM((nb * qpw * _K,), jnp.int32),
            pltpu.VMEM((nb * tw * qpw,), jnp.float32),
            pltpu.VMEM((nb * tw * qpw,), jnp.float32),
            pltpu.VMEM((nb * qpw * tw,), jnp.float32),
            pltpu.VMEM((nb * qpw * tw,), jnp.float32),
            pltpu.VMEM((nb * qpw * 3,), jnp.float32),
            pltpu.VMEM((nb * qpw,), jnp.int32),
            pltpu.VMEM((24,), jnp.float32),
            pltpu.SemaphoreType.DMA,
        ],
    )
    def knn(xs_hbm, ys_hbm, xb_hbm, yb_hbm, data_hbm, labels_hbm, tst_hbm,
            edge_hbm, u_hbm, yl_hbm, pos_hbm, bat_hbm,
            xs_v, ys_v, xb_v, yb_v, sq_v, ji_v, out_v, src_v, dst_v,
            dsl_v, lsl_v, ust_v, yst_v, pst_v, bst_v, tst_v, sem):
        wid = lax.axis_index("s") * _NC + lax.axis_index("c")
        base = wid * qpw

        loads = []
        for b in range(nb):
            for t in range(tw):
                o = (b * tw + t) * qpw
                loads.append(pltpu.async_copy(
                    data_hbm.at[b, t, pl.ds(base, qpw)],
                    dsl_v.at[pl.ds(o, qpw)], sem))
                loads.append(pltpu.async_copy(
                    labels_hbm.at[b, t, pl.ds(base, qpw)],
                    lsl_v.at[pl.ds(o, qpw)], sem))
        loads.append(pltpu.async_copy(tst_hbm, tst_v, sem))

        pltpu.sync_copy(xs_hbm, xs_v.at[pl.ds(0, nx)])
        pltpu.sync_copy(ys_hbm, ys_v.at[pl.ds(0, nx)])
        pltpu.sync_copy(xb_hbm, xb_v.at[pl.ds(0, nx)])
        pltpu.sync_copy(yb_hbm, yb_v.at[pl.ds(0, nx)])

        ramp = lax.broadcasted_iota(jnp.int32, (_L,), 0)

        def sq_body(g, carry):
            xv = xs_v[pl.ds(g * _L, _L)]
            yv = ys_v[pl.ds(g * _L, _L)]
            sq_v[pl.ds(g * _L, _L)] = xv * xv + yv * yv
            ji_v[pl.ds(g * _L, _L)] = ramp + g * _L
            return carry

        lax.fori_loop(0, ngroups, sq_body, 0)

        qb = 4

        def q_body(qblk, carry):
            nxi, nyi, sqi, ivs = [], [], [], []
            for k in range(qb):
                i = base + qblk * qb + k
                nxi.append(jnp.full((_L,), xb_v[pl.ds(i, _L)][0],
                                    jnp.float32) * -2.0)
                nyi.append(jnp.full((_L,), yb_v[pl.ds(i, _L)][0],
                                    jnp.float32) * -2.0)
                sqi.append(jnp.full((_L,), sq_v[pl.ds(i, _L)][0],
                                    jnp.float32))
                ivs.append(jnp.full((_L,), i, jnp.int32))

            def c_body(g, bst):
                jx = xb_v[pl.ds(g * _L, _L)]
                jy = yb_v[pl.ds(g * _L, _L)]
                jq = sq_v[pl.ds(g * _L, _L)]
                jv = ji_v[pl.ds(g * _L, _L)]
                out = []
                for k in range(qb):
                    bd, bi = bst[2 * k], bst[2 * k + 1]
                    d = (sqi[k] + jq) + (nxi[k] * jx + nyi[k] * jy)
                    d = jnp.where(jv == ivs[k], d + 1e10, d)
                    sd, si = plsc.sort_key_val(d, jv)
                    sdr = lax.rev(sd, (0,))
                    sir = lax.rev(si, (0,))
                    take = sdr < bd
                    ld = jnp.where(take, sdr, bd)
                    li = jnp.where(take, sir, bi)
                    nd, ni = plsc.sort_key_val(ld, li)
                    out += [nd, ni]
                return tuple(out)

            init = []
            for k in range(qb):
                init += [jnp.full((_L,), 3e38, jnp.float32),
                         jnp.zeros((_L,), jnp.int32)]
            bst = lax.fori_loop(0, ngroups, c_body, tuple(init))
            for k in range(qb):
                q = qblk * qb + k
                out_v[pl.ds(q * _K, _K)] = bst[2 * k + 1]
            return carry

        lax.fori_loop(0, qpw // qb, q_body, 0)

        nqk = qpw * _K
        stores = []

        def edge_body(c, carry):
            loc = out_v[pl.ds(c * _L, _L)]
            for b in range(nb):
                src_v[pl.ds(b * nqk + c * _L, _L)] = loc + b * nx
            return carry

        lax.fori_loop(0, nqk // _L, edge_body, 0)

        def dstq_body(q, carry):
            for b in range(nb):
                dst_v[pl.ds(b * nqk + q * _K, _K)] = (
                    jnp.full((_L,), b * nx + base, jnp.int32) + q)
            return carry

        lax.fori_loop(0, qpw, dstq_body, 0)

        for b in range(nb):
            stores.append(pltpu.async_copy(
                src_v.at[pl.ds(b * nqk, nqk)],
                edge_hbm.at[pl.ds(b * nx * _K + base * _K, nqk)], sem))
            stores.append(pltpu.async_copy(
                dst_v.at[pl.ds(b * nqk, nqk)],
                edge_hbm.at[pl.ds((nb + b) * nx * _K + base * _K, nqk)],
                sem))

        for cp in loads:
            cp.wait()

        ramp_tw = ramp * tw
        nchunks = qpw // _L

        def slab_body(bt, carry):
            b = bt // tw
            t = bt % tw
            src_off = (b * tw + t) * qpw
            for c in range(nchunks):
                off = b * (qpw * tw) + c * (_L * tw) + t
                plsc.store_scatter(ust_v, [ramp_tw + off],
                                   dsl_v[pl.ds(src_off + c * _L, _L)])
                plsc.store_scatter(yst_v, [ramp_tw + off],
                                   lsl_v[pl.ds(src_off + c * _L, _L)])
            return carry

        lax.fori_loop(0, nb * tw, slab_body, 0)

        ramp3 = ramp * 3

        def pos_body(b, carry):
            tb = jnp.full((_L,), tst_v[pl.ds(b, _L)][0], jnp.float32)
            bv = jnp.full((_L,), b, jnp.int32)
            for c in range(nchunks):
                off = b * (qpw * 3) + c * (_L * 3)
                plsc.store_scatter(pst_v, [ramp3 + off], tb)
                plsc.store_scatter(pst_v, [ramp3 + (off + 1)],
                                   xs_v[pl.ds(base + c * _L, _L)])
                plsc.store_scatter(pst_v, [ramp3 + (off + 2)],
                                   ys_v[pl.ds(base + c * _L, _L)])
                bst_v[pl.ds(b * qpw + c * _L, _L)] = bv
            return carry

        lax.fori_loop(0, nb, pos_body, 0)

        for b in range(nb):
            stores.append(pltpu.async_copy(
                ust_v.at[pl.ds(b * qpw * tw, qpw * tw)],
                u_hbm.at[pl.ds((b * nx + base) * tw, qpw * tw)], sem))
            stores.append(pltpu.async_copy(
                yst_v.at[pl.ds(b * qpw * tw, qpw * tw)],
                yl_hbm.at[pl.ds((b * nx + base) * tw, qpw * tw)], sem))
            stores.append(pltpu.async_copy(
                pst_v.at[pl.ds(b * qpw * 3, qpw * 3)],
                pos_hbm.at[pl.ds((b * nx + base) * 3, qpw * 3)], sem))
            stores.append(pltpu.async_copy(
                bst_v.at[pl.ds(b * qpw, qpw)],
                bat_hbm.at[pl.ds(b * nx + base, qpw)], sem))
        for cp in stores:
            cp.wait()

    return knn(xs, ys, xb, yb, data, labels, tstep)


def kernel(data, labels, x, nu, steps):
    B, tw, nx = data.shape
    nt = _NS_STEP
    tmax = _NS_STEP * _NS_DT
    t = jnp.linspace(0.0, tmax, nt)
    tstep = jnp.pad(t[steps], (0, 24 - B))

    x0 = x[0]
    xb = lax.reduce_precision(x0, exponent_bits=8, mantissa_bits=7)
    edge, u, y, pos, batch = _graph_sc(
        x0[:, 0], x0[:, 1], xb[:, 0], xb[:, 1], data, labels, tstep)

    edge_index = edge.reshape(2, B * nx * _K)
    u = u.reshape(B * nx, tw)
    y = y.reshape(B * nx, tw)
    pos = pos.reshape(B * nx, 3)
    parameters = nu
    return (u, edge_index, y, pos, batch, parameters)


# speedup vs baseline: 1.8766x; 1.8766x over previous
import functools

import jax
import jax.numpy as jnp
from jax import lax
from jax.experimental import pallas as pl
from jax.experimental.pallas import tpu as pltpu
from jax.experimental.pallas import tpu_sc as plsc

_NS_DT = 0.1
_NS_STEP = 40
_K = 16
_L = 16
_NC = 2
_NSUB = 16


def _knn_edges_sc(xs, ys, xb, yb, nbatch):
    nx = xs.shape[0]
    nw = _NC * _NSUB
    qpw = nx // nw
    ngroups = nx // _L
    mesh = plsc.VectorSubcoreMesh(core_axis_name="c", subcore_axis_name="s")

    @functools.partial(
        pl.kernel,
        out_type=jax.ShapeDtypeStruct((2 * nbatch * nx * _K,), jnp.int32),
        mesh=mesh,
        compiler_params=pltpu.CompilerParams(needs_layout_passes=False),
        scratch_types=[
            pltpu.VMEM((nx + _L,), jnp.float32),
            pltpu.VMEM((nx + _L,), jnp.float32),
            pltpu.VMEM((nx + _L,), jnp.float32),
            pltpu.VMEM((nx + _L,), jnp.float32),
            pltpu.VMEM((nx + _L,), jnp.float32),
            pltpu.VMEM((nx + _L,), jnp.int32),
            pltpu.VMEM((qpw * _K,), jnp.int32),
            pltpu.VMEM((qpw * _K,), jnp.int32),
            pltpu.VMEM((qpw * _K,), jnp.int32),
        ],
    )
    def knn(xs_hbm, ys_hbm, xb_hbm, yb_hbm, out_hbm,
            xs_v, ys_v, xb_v, yb_v, sq_v, ji_v, out_v, dst_v, tmp_v):
        wid = lax.axis_index("s") * _NC + lax.axis_index("c")
        pltpu.sync_copy(xs_hbm, xs_v.at[pl.ds(0, nx)])
        pltpu.sync_copy(ys_hbm, ys_v.at[pl.ds(0, nx)])
        pltpu.sync_copy(xb_hbm, xb_v.at[pl.ds(0, nx)])
        pltpu.sync_copy(yb_hbm, yb_v.at[pl.ds(0, nx)])

        ramp = lax.broadcasted_iota(jnp.int32, (_L,), 0)

        def sq_body(g, carry):
            xv = xs_v[pl.ds(g * _L, _L)]
            yv = ys_v[pl.ds(g * _L, _L)]
            sq_v[pl.ds(g * _L, _L)] = xv * xv + yv * yv
            ji_v[pl.ds(g * _L, _L)] = ramp + g * _L
            return carry

        lax.fori_loop(0, ngroups, sq_body, 0)

        base = wid * qpw
        qb = 4

        def q_body(qblk, carry):
            iq, nxi, nyi, sqi, ivs = [], [], [], [], []
            for k in range(qb):
                i = base + qblk * qb + k
                iq.append(i)
                nxi.append(jnp.full((_L,), xb_v[pl.ds(i, _L)][0],
                                    jnp.float32) * -2.0)
                nyi.append(jnp.full((_L,), yb_v[pl.ds(i, _L)][0],
                                    jnp.float32) * -2.0)
                sqi.append(jnp.full((_L,), sq_v[pl.ds(i, _L)][0],
                                    jnp.float32))
                ivs.append(jnp.full((_L,), i, jnp.int32))

            def c_body(g, bst):
                jx = xb_v[pl.ds(g * _L, _L)]
                jy = yb_v[pl.ds(g * _L, _L)]
                jq = sq_v[pl.ds(g * _L, _L)]
                jv = ji_v[pl.ds(g * _L, _L)]
                out = []
                for k in range(qb):
                    bd, bi = bst[2 * k], bst[2 * k + 1]
                    d = (sqi[k] + jq) + (nxi[k] * jx + nyi[k] * jy)
                    d = jnp.where(jv == ivs[k], jnp.float32(3e38), d)
                    sd, si = plsc.sort_key_val(d, jv)
                    take = sd < bd
                    nd = jnp.minimum(bd, sd)
                    ni = jnp.where(take, si, bi)
                    nd, ni = plsc.sort_key_val(nd, ni, descending=True)
                    out += [nd, ni]
                return tuple(out)

            init = []
            for k in range(qb):
                init += [jnp.full((_L,), 3e38, jnp.float32),
                         jnp.zeros((_L,), jnp.int32)]
            bst = lax.fori_loop(0, ngroups, c_body, tuple(init))
            for k in range(qb):
                q = qblk * qb + k
                _, fi = plsc.sort_key_val(bst[2 * k], bst[2 * k + 1])
                out_v[pl.ds(q * _K, _K)] = fi
                dst_v[pl.ds(q * _K, _K)] = jnp.full((_L,), base + q,
                                                    jnp.int32)
            return carry

        lax.fori_loop(0, qpw // qb, q_body, 0)

        nqk = qpw * _K

        def off_body(c, boff):
            tmp_v[pl.ds(c * _L, _L)] = out_v[pl.ds(c * _L, _L)] + boff
            return boff

        def dst_body(c, boff):
            tmp_v[pl.ds(c * _L, _L)] = dst_v[pl.ds(c * _L, _L)] + boff
            return boff

        for b in range(nbatch):
            lax.fori_loop(0, nqk // _L, off_body, jnp.int32(b * nx))
            pltpu.sync_copy(
                tmp_v, out_hbm.at[pl.ds(b * nx * _K + base * _K, nqk)])
        for b in range(nbatch):
            lax.fori_loop(0, nqk // _L, dst_body, jnp.int32(b * nx))
            pltpu.sync_copy(
                tmp_v,
                out_hbm.at[pl.ds((nbatch + b) * nx * _K + base * _K, nqk)])

    return knn(xs, ys, xb, yb).reshape(2, nbatch * nx * _K)


def kernel(data, labels, x, nu, steps):
    B, tw, nx = data.shape
    nt = _NS_STEP
    tmax = _NS_STEP * _NS_DT
    t = jnp.linspace(0.0, tmax, nt)

    u = jnp.transpose(data, (0, 2, 1)).reshape(B * nx, tw)
    y = jnp.transpose(labels, (0, 2, 1)).reshape(B * nx, tw)

    x0 = x[0]
    xb = lax.reduce_precision(x0, exponent_bits=8, mantissa_bits=7)
    edge_index = _knn_edges_sc(x0[:, 0], x0[:, 1], xb[:, 0], xb[:, 1], B)

    x_pos = jnp.tile(x0, (B, 1))
    t_pos = jnp.repeat(t[steps], nx)[:, None]
    pos = jnp.concatenate([t_pos, x_pos], axis=-1)
    batch = jnp.repeat(jnp.arange(B, dtype=jnp.int32), nx)
    parameters = nu
    return (u, edge_index, y, pos, batch, parameters)
